# baseline (speedup 1.0000x reference)
import jax
import jax.numpy as jnp
from jax.experimental import pallas as pl
from jax.experimental.pallas import tpu as pltpu

EPS = 1e-7
TILE = 2048


def _zero_col0(m):
    col = jax.lax.broadcasted_iota(jnp.int32, m.shape, 1)
    return jnp.where(col == 0, 0.0, m)


def _roundtrip(u):
    n2 = jnp.sum(u * u, axis=-1, keepdims=True)
    n = jnp.sqrt(jnp.maximum(n2, EPS))
    e = jnp.exp(n)
    ei = jnp.exp(-n)
    sinh_n = (e - ei) * 0.5
    cosh_n = (e + ei) * 0.5
    sp = (sinh_n / n) * u
    t = jnp.maximum(cosh_n, 1.0 + EPS)
    r = jnp.log(t + jnp.sqrt(t + 1.0) * jnp.sqrt(t - 1.0))
    ns2 = jnp.sum(sp * sp, axis=-1, keepdims=True)
    ns = jnp.sqrt(jnp.maximum(ns2, EPS))
    return (r / ns) * sp


def _bdot(a, b):
    return jnp.dot(
        a.astype(jnp.bfloat16),
        b.astype(jnp.bfloat16),
        preferred_element_type=jnp.float32,
    )


def _fused_kernel(
    adj_ref, x_ref, we_ref, w0_ref, b0_ref, w1_ref, b1_ref, wml_ref, bml_ref,
    mask_ref, out_ref,
):
    adj_bf = adj_ref[0].astype(jnp.bfloat16)
    h = _bdot(x_ref[0], we_ref[...])
    h = _zero_col0(h)
    g1 = _bdot(h, w0_ref[...]) + b0_ref[...]
    a1 = jnp.dot(adj_bf, g1.astype(jnp.bfloat16), preferred_element_type=jnp.float32)
    u1 = _zero_col0(jnp.maximum(a1, 0.0))
    v1 = _roundtrip(u1)
    g2 = _bdot(v1, w1_ref[...]) + b1_ref[...]
    a2 = jnp.dot(adj_bf, g2.astype(jnp.bfloat16), preferred_element_type=jnp.float32)
    u2 = _zero_col0(jnp.maximum(a2, 0.0))
    v2 = _roundtrip(u2)
    out_ref[0] = (_bdot(v2, wml_ref[...]) + bml_ref[...]) * mask_ref[0]


def kernel(x, adj, node_mask, W_emb, W0, b0, W1, b1, W_ml, b_ml):
    B, N, F_IN = x.shape
    H = W0.shape[0]
    OUT = W_ml.shape[1]
    b0r = b0.reshape(1, H)
    b1r = b1.reshape(1, H)
    bmlr = b_ml.reshape(1, OUT)

    out = pl.pallas_call(
        _fused_kernel,
        grid=(B,),
        in_specs=[
            pl.BlockSpec((1, N, N), lambda b: (b, 0, 0)),
            pl.BlockSpec((1, N, F_IN), lambda b: (b, 0, 0)),
            pl.BlockSpec((F_IN, H), lambda b: (0, 0)),
            pl.BlockSpec((H, H), lambda b: (0, 0)),
            pl.BlockSpec((1, H), lambda b: (0, 0)),
            pl.BlockSpec((H, H), lambda b: (0, 0)),
            pl.BlockSpec((1, H), lambda b: (0, 0)),
            pl.BlockSpec((H, OUT), lambda b: (0, 0)),
            pl.BlockSpec((1, OUT), lambda b: (0, 0)),
            pl.BlockSpec((1, N, 1), lambda b: (b, 0, 0)),
        ],
        out_specs=pl.BlockSpec((1, N, OUT), lambda b: (b, 0, 0)),
        out_shape=jax.ShapeDtypeStruct((B, N, OUT), jnp.float32),
        compiler_params=pltpu.CompilerParams(
            dimension_semantics=("parallel",),
        ),
    )(adj, x, W_emb, W0, b0r, W1, b1r, W_ml, bmlr, node_mask)

    return out


# speedup vs baseline: 1.2216x; 1.2216x over previous
import jax
import jax.numpy as jnp
from jax.experimental import pallas as pl
from jax.experimental.pallas import tpu as pltpu

EPS = 1e-7
TILE = 2048


def _zero_col0(m):
    col = jax.lax.broadcasted_iota(jnp.int32, m.shape, 1)
    return jnp.where(col == 0, 0.0, m)


def _roundtrip_t(uT):
    n2 = jnp.sum(uT * uT, axis=0, keepdims=True)
    n = jnp.sqrt(jnp.maximum(n2, EPS))
    e = jnp.exp(n)
    ei = jnp.exp(-n)
    sinh_n = (e - ei) * 0.5
    cosh_n = (e + ei) * 0.5
    sp = (sinh_n / n) * uT
    t = jnp.maximum(cosh_n, 1.0 + EPS)
    r = jnp.log(t + jnp.sqrt(t + 1.0) * jnp.sqrt(t - 1.0))
    ns2 = jnp.sum(sp * sp, axis=0, keepdims=True)
    ns = jnp.sqrt(jnp.maximum(ns2, EPS))
    return (r / ns) * sp


def _bdot(a, b):
    return jnp.dot(
        a.astype(jnp.bfloat16),
        b.astype(jnp.bfloat16),
        preferred_element_type=jnp.float32,
    )


def _fused_kernel(
    adj_ref, x_ref, we_ref, w0_ref, b0_ref, w1_ref, b1_ref, wml_ref, bml_ref,
    mask_ref, out_ref,
):
    adj_bf = adj_ref[0].astype(jnp.bfloat16)
    h = _bdot(x_ref[0], we_ref[...])
    h = _zero_col0(h)
    g1 = _bdot(h, w0_ref[...]) + b0_ref[...]
    a1 = jnp.dot(adj_bf, g1.astype(jnp.bfloat16), preferred_element_type=jnp.float32)
    u1T = jnp.transpose(_zero_col0(jnp.maximum(a1, 0.0)))
    v1T = _roundtrip_t(u1T)
    g2 = (
        jax.lax.dot_general(
            v1T.astype(jnp.bfloat16),
            w1_ref[...].astype(jnp.bfloat16),
            (((0,), (0,)), ((), ())),
            preferred_element_type=jnp.float32,
        )
        + b1_ref[...]
    )
    a2 = jnp.dot(adj_bf, g2.astype(jnp.bfloat16), preferred_element_type=jnp.float32)
    u2T = jnp.transpose(_zero_col0(jnp.maximum(a2, 0.0)))
    v2T = _roundtrip_t(u2T)
    ml = (
        jax.lax.dot_general(
            v2T.astype(jnp.bfloat16),
            wml_ref[...].astype(jnp.bfloat16),
            (((0,), (0,)), ((), ())),
            preferred_element_type=jnp.float32,
        )
        + bml_ref[...]
    )
    out_ref[0] = ml * mask_ref[0]


def kernel(x, adj, node_mask, W_emb, W0, b0, W1, b1, W_ml, b_ml):
    B, N, F_IN = x.shape
    H = W0.shape[0]
    OUT = W_ml.shape[1]
    b0r = b0.reshape(1, H)
    b1r = b1.reshape(1, H)
    bmlr = b_ml.reshape(1, OUT)

    out = pl.pallas_call(
        _fused_kernel,
        grid=(B,),
        in_specs=[
            pl.BlockSpec((1, N, N), lambda b: (b, 0, 0)),
            pl.BlockSpec((1, N, F_IN), lambda b: (b, 0, 0)),
            pl.BlockSpec((F_IN, H), lambda b: (0, 0)),
            pl.BlockSpec((H, H), lambda b: (0, 0)),
            pl.BlockSpec((1, H), lambda b: (0, 0)),
            pl.BlockSpec((H, H), lambda b: (0, 0)),
            pl.BlockSpec((1, H), lambda b: (0, 0)),
            pl.BlockSpec((H, OUT), lambda b: (0, 0)),
            pl.BlockSpec((1, OUT), lambda b: (0, 0)),
            pl.BlockSpec((1, N, 1), lambda b: (b, 0, 0)),
        ],
        out_specs=pl.BlockSpec((1, N, OUT), lambda b: (b, 0, 0)),
        out_shape=jax.ShapeDtypeStruct((B, N, OUT), jnp.float32),
        compiler_params=pltpu.CompilerParams(
            dimension_semantics=("parallel",),
        ),
    )(adj, x, W_emb, W0, b0r, W1, b1r, W_ml, bmlr, node_mask)

    return out


# speedup vs baseline: 1.2229x; 1.0011x over previous
import jax
import jax.numpy as jnp
from jax.experimental import pallas as pl
from jax.experimental.pallas import tpu as pltpu

EPS = 1e-7


def _zero_col0(m):
    col = jax.lax.broadcasted_iota(jnp.int32, m.shape, 1)
    return jnp.where(col == 0, 0.0, m)


def _roundtrip_t(uT):
    n2 = jnp.sum(uT * uT, axis=0, keepdims=True)
    n = jnp.sqrt(jnp.maximum(n2, EPS))
    e = jnp.exp(n)
    ei = jnp.exp(-n)
    sinh_n = (e - ei) * 0.5
    cosh_n = (e + ei) * 0.5
    sp = (sinh_n / n) * uT
    t = jnp.maximum(cosh_n, 1.0 + EPS)
    r = jnp.log(t + jnp.sqrt(t + 1.0) * jnp.sqrt(t - 1.0))
    ns2 = jnp.sum(sp * sp, axis=0, keepdims=True)
    ns = jnp.sqrt(jnp.maximum(ns2, EPS))
    return (r / ns) * sp


def _bdot(a, b):
    return jnp.dot(
        a.astype(jnp.bfloat16),
        b.astype(jnp.bfloat16),
        preferred_element_type=jnp.float32,
    )


def _fused_kernel(
    adj_ref, x_ref, we_ref, w0_ref, b0_ref, w1_ref, b1_ref, wml_ref, bml_ref,
    mask_ref, out_ref,
):
    adj_bf = adj_ref[0].astype(jnp.bfloat16)
    h = _bdot(x_ref[0], we_ref[...])
    h = _zero_col0(h)
    g1 = _bdot(h, w0_ref[...]) + b0_ref[...]
    a1 = jnp.dot(adj_bf, g1.astype(jnp.bfloat16), preferred_element_type=jnp.float32)
    u1T = jnp.transpose(_zero_col0(jnp.maximum(a1, 0.0)))
    v1T = _roundtrip_t(u1T)
    g2 = (
        jax.lax.dot_general(
            v1T.astype(jnp.bfloat16),
            w1_ref[...].astype(jnp.bfloat16),
            (((0,), (0,)), ((), ())),
            preferred_element_type=jnp.float32,
        )
        + b1_ref[...]
    )
    a2 = jnp.dot(adj_bf, g2.astype(jnp.bfloat16), preferred_element_type=jnp.float32)
    u2T = jnp.transpose(_zero_col0(jnp.maximum(a2, 0.0)))
    v2T = _roundtrip_t(u2T)
    ml = (
        jax.lax.dot_general(
            v2T.astype(jnp.bfloat16),
            wml_ref[...].astype(jnp.bfloat16),
            (((0,), (0,)), ((), ())),
            preferred_element_type=jnp.float32,
        )
        + bml_ref[...]
    )
    out_ref[0] = ml * mask_ref[0]


def kernel(x, adj, node_mask, W_emb, W0, b0, W1, b1, W_ml, b_ml):
    B, N, F_IN = x.shape
    H = W0.shape[0]
    OUT = W_ml.shape[1]
    b0r = b0.reshape(1, H)
    b1r = b1.reshape(1, H)
    bmlr = b_ml.reshape(1, OUT)

    out = pl.pallas_call(
        _fused_kernel,
        grid=(B,),
        in_specs=[
            pl.BlockSpec((1, N, N), lambda b: (b, 0, 0)),
            pl.BlockSpec((1, N, F_IN), lambda b: (b, 0, 0)),
            pl.BlockSpec((F_IN, H), lambda b: (0, 0)),
            pl.BlockSpec((H, H), lambda b: (0, 0)),
            pl.BlockSpec((1, H), lambda b: (0, 0)),
            pl.BlockSpec((H, H), lambda b: (0, 0)),
            pl.BlockSpec((1, H), lambda b: (0, 0)),
            pl.BlockSpec((H, OUT), lambda b: (0, 0)),
            pl.BlockSpec((1, OUT), lambda b: (0, 0)),
            pl.BlockSpec((1, N, 1), lambda b: (b, 0, 0)),
        ],
        out_specs=pl.BlockSpec((1, N, OUT), lambda b: (b, 0, 0)),
        out_shape=jax.ShapeDtypeStruct((B, N, OUT), jnp.float32),
        compiler_params=pltpu.CompilerParams(
            dimension_semantics=("parallel",),
        ),
    )(adj, x, W_emb, W0, b0r, W1, b1r, W_ml, bmlr, node_mask)

    return out


# speedup vs baseline: 1.2231x; 1.0001x over previous
import jax
import jax.numpy as jnp
from jax.experimental import pallas as pl
from jax.experimental.pallas import tpu as pltpu

EPS = 1e-7


def _zero_col0(m):
    col = jax.lax.broadcasted_iota(jnp.int32, m.shape, 1)
    return jnp.where(col == 0, 0.0, m)


def _roundtrip_t(uT):
    n2 = jnp.sum(uT * uT, axis=0, keepdims=True)
    n = jnp.sqrt(jnp.maximum(n2, EPS))
    ln2 = 0.6931471805599453
    e = jnp.exp(n - ln2)
    ei = jnp.exp(-n - ln2)
    sinh_n = e - ei
    cosh_n = e + ei
    sp = (sinh_n / n) * uT
    t = jnp.maximum(cosh_n, 1.0 + EPS)
    r = jnp.log(t) + jnp.log1p(jnp.sqrt(1.0 - jnp.square(1.0 / t)))
    ns2 = jnp.sum(sp * sp, axis=0, keepdims=True)
    ns = jnp.sqrt(jnp.maximum(ns2, EPS))
    return (r / ns) * sp


def _bdot(a, b):
    return jnp.dot(
        a.astype(jnp.bfloat16),
        b.astype(jnp.bfloat16),
        preferred_element_type=jnp.float32,
    )


def _fused_kernel(
    adj_ref, x_ref, we_ref, w0_ref, b0_ref, w1_ref, b1_ref, wml_ref, bml_ref,
    mask_ref, out_ref,
):
    adj_bf = adj_ref[0].astype(jnp.bfloat16)
    h = _bdot(x_ref[0], we_ref[...])
    h = _zero_col0(h)
    g1 = _bdot(h, w0_ref[...]) + b0_ref[...]
    a1 = jnp.dot(adj_bf, g1.astype(jnp.bfloat16), preferred_element_type=jnp.float32)
    u1T = jnp.transpose(_zero_col0(jnp.maximum(a1, 0.0)))
    v1T = _roundtrip_t(u1T)
    g2 = (
        jax.lax.dot_general(
            v1T.astype(jnp.bfloat16),
            w1_ref[...].astype(jnp.bfloat16),
            (((0,), (0,)), ((), ())),
            preferred_element_type=jnp.float32,
        )
        + b1_ref[...]
    )
    a2 = jnp.dot(adj_bf, g2.astype(jnp.bfloat16), preferred_element_type=jnp.float32)
    u2T = jnp.transpose(_zero_col0(jnp.maximum(a2, 0.0)))
    v2T = _roundtrip_t(u2T)
    ml = (
        jax.lax.dot_general(
            v2T.astype(jnp.bfloat16),
            wml_ref[...].astype(jnp.bfloat16),
            (((0,), (0,)), ((), ())),
            preferred_element_type=jnp.float32,
        )
        + bml_ref[...]
    )
    out_ref[0] = ml * mask_ref[0]


def kernel(x, adj, node_mask, W_emb, W0, b0, W1, b1, W_ml, b_ml):
    B, N, F_IN = x.shape
    H = W0.shape[0]
    OUT = W_ml.shape[1]
    b0r = b0.reshape(1, H)
    b1r = b1.reshape(1, H)
    bmlr = b_ml.reshape(1, OUT)

    out = pl.pallas_call(
        _fused_kernel,
        grid=(B,),
        in_specs=[
            pl.BlockSpec((1, N, N), lambda b: (b, 0, 0)),
            pl.BlockSpec((1, N, F_IN), lambda b: (b, 0, 0)),
            pl.BlockSpec((F_IN, H), lambda b: (0, 0)),
            pl.BlockSpec((H, H), lambda b: (0, 0)),
            pl.BlockSpec((1, H), lambda b: (0, 0)),
            pl.BlockSpec((H, H), lambda b: (0, 0)),
            pl.BlockSpec((1, H), lambda b: (0, 0)),
            pl.BlockSpec((H, OUT), lambda b: (0, 0)),
            pl.BlockSpec((1, OUT), lambda b: (0, 0)),
            pl.BlockSpec((1, N, 1), lambda b: (b, 0, 0)),
        ],
        out_specs=pl.BlockSpec((1, N, OUT), lambda b: (b, 0, 0)),
        out_shape=jax.ShapeDtypeStruct((B, N, OUT), jnp.float32),
        compiler_params=pltpu.CompilerParams(
            dimension_semantics=("parallel",),
        ),
    )(adj, x, W_emb, W0, b0r, W1, b1r, W_ml, bmlr, node_mask)

    return out
